# baseline (speedup 1.0000x reference)
import functools

import jax
import jax.numpy as jnp
from jax import lax
from jax.experimental import pallas as pl
from jax.experimental.pallas import tpu as pltpu
from jax.experimental.pallas import tpu_sc as plsc

_BATCH = 16384
_D = 32
_RANK = 16
_NC = 2
_NS = 16
_NW = _NC * _NS
_BPW = _BATCH // _NW
_G = _BPW // 16


def _mf_body(users_hbm, items_hbm, wu_hbm, wi_hbm, a_hbm, ub_hbm, ib_hbm,
             b_hbm, out_hbm,
             uidx_v, iidx_v, u_v, wi_v, a_v, ubias_v, ibias_v, b_v, pred_v,
             sem):
    wid = lax.axis_index("s") * _NC + lax.axis_index("c")
    base = wid * _BPW
    pltpu.sync_copy(users_hbm.at[pl.ds(base, _BPW)], uidx_v)
    pltpu.sync_copy(items_hbm.at[pl.ds(base, _BPW)], iidx_v)
    pltpu.sync_copy(b_hbm, b_v)

    cps = [
        pltpu.async_copy(wu_hbm.at[uidx_v], u_v, sem),
        pltpu.async_copy(wi_hbm.at[iidx_v], wi_v, sem),
        pltpu.async_copy(a_hbm.at[iidx_v], a_v, sem),
        pltpu.async_copy(ub_hbm.at[uidx_v], ubias_v, sem),
        pltpu.async_copy(ib_hbm.at[iidx_v], ibias_v, sem),
    ]
    for c in cps:
        c.wait()

    brows = [(b_v[r, pl.ds(0, 16)], b_v[r, pl.ds(16, 16)]) for r in range(_RANK)]
    brows_half = [0] * 16 + [1] * 16
    lanes = lax.iota(jnp.int32, 16)

    def group(g, carry):
        s = g * 16
        rows = s + lanes
        acc = ubias_v[pl.ds(s, 16)] + ibias_v[pl.ds(s, 16)]
        vr = [jnp.zeros((16,), jnp.float32) for _ in range(_RANK)]
        for d in range(_D):
            col = jnp.full((16,), d, jnp.int32)
            uc = plsc.load_gather(u_v, [rows, col])
            wc = plsc.load_gather(wi_v, [rows, col])
            acc = acc + uc * wc
            half = brows_half[d]
            lane = d % 16
            for r in range(_RANK):
                vr[r] = vr[r] + brows[r][half][lane] * uc
        for r in range(_RANK):
            ac = plsc.load_gather(a_v, [rows, jnp.full((16,), r, jnp.int32)])
            acc = acc + ac * vr[r]
        pred_v[pl.ds(s, 16)] = acc
        return carry

    lax.fori_loop(0, _G, group, 0)
    pltpu.sync_copy(pred_v, out_hbm.at[pl.ds(base, _BPW)])


_sc_mf = functools.partial(
    pl.kernel,
    out_type=jax.ShapeDtypeStruct((_BATCH,), jnp.float32),
    mesh=plsc.VectorSubcoreMesh(core_axis_name="c", subcore_axis_name="s"),
    compiler_params=pltpu.CompilerParams(use_tc_tiling_on_sc=False,
                                         needs_layout_passes=False),
    scratch_types=[
        pltpu.VMEM((_BPW,), jnp.int32),
        pltpu.VMEM((_BPW,), jnp.int32),
        pltpu.VMEM((_BPW, _D), jnp.float32),
        pltpu.VMEM((_BPW, _D), jnp.float32),
        pltpu.VMEM((_BPW, _RANK), jnp.float32),
        pltpu.VMEM((_BPW,), jnp.float32),
        pltpu.VMEM((_BPW,), jnp.float32),
        pltpu.VMEM((_RANK, _D), jnp.float32),
        pltpu.VMEM((_BPW,), jnp.float32),
        pltpu.SemaphoreType.DMA,
    ],
)(_mf_body)


def kernel(users, items, W_user, W_item, user_bias, item_bias, A, B):
    users = users.astype(jnp.int32)
    items = items.astype(jnp.int32)
    return _sc_mf(users, items, W_user, W_item, A, user_bias, item_bias, B)
